# baseline (speedup 1.0000x reference)
import functools

import jax
import jax.numpy as jnp
from jax.experimental import pallas as pl
from jax.experimental.pallas import tpu as pltpu
from jax.experimental.pallas import tpu_sc as plsc

def _search_kernel(pt_ref, g_ref, pth_ref, gh2x_ref,
                   idx_pg_ref, idx_gp_ref, colmin_ref, colidx_ref,
                   *, n_row_tiles, tm, n, m):
    b = pl.program_id(0)
    i = pl.program_id(1)

    p = pt_ref[0]
    g = g_ref[0]
    px = p[:, 0:1]
    py = p[:, 1:2]
    pz = p[:, 2:3]
    gx = g[0:1, :]
    gy = g[1:2, :]
    gz = g[2:3, :]

    p2 = px * px + py * py + pz * pz
    g2 = gx * gx + gy * gy + gz * gz
    s = p2 + g2

    cross2_sel = jnp.dot(pth_ref[0], gh2x_ref[0],
                         preferred_element_type=jnp.float32)
    d_sel = s - cross2_sel

    rowidx = jnp.argmin(d_sel, axis=1).astype(jnp.int32)[:, None]
    idx_pg_ref[0] = rowidx + b * n

    tile_colmin = jnp.min(d_sel, axis=0, keepdims=True)
    tile_colidx = (jnp.argmin(d_sel, axis=0).astype(jnp.int32)[None, :]
                   + i * tm)

    @pl.when(i == 0)
    def _init_col():
        colmin_ref[...] = tile_colmin
        colidx_ref[...] = tile_colidx

    @pl.when(i > 0)
    def _update_col():
        better = tile_colmin < colmin_ref[...]
        colmin_ref[...] = jnp.where(better, tile_colmin, colmin_ref[...])
        colidx_ref[...] = jnp.where(better, tile_colidx, colidx_ref[...])

    @pl.when(i == n_row_tiles - 1)
    def _finish_batch():
        idx_gp_ref[0] = colidx_ref[...] + b * m


def _reduce_kernel(self_f_ref, qf_ref, sel_b_ref, qb_ref, out_ref, acc_ref,
                   *, count, nsteps):
    k = pl.program_id(0)
    diff_f = self_f_ref[...][:, 0:3] - qf_ref[...]
    diff_b = sel_b_ref[...][:, 0:3] - qb_ref[...]
    rs_f = jnp.sum(diff_f * diff_f, axis=1, keepdims=True)
    rs_b = jnp.sum(diff_b * diff_b, axis=1, keepdims=True)
    part = (jnp.sum(jnp.sqrt(rs_f + 1e-8)) + jnp.sum(jnp.sqrt(rs_b + 1e-8)))

    @pl.when(k == 0)
    def _init():
        acc_ref[0, 0] = 0.0

    acc_ref[0, 0] += part

    @pl.when(k == nsteps - 1)
    def _finish():
        out_ref[...] = jnp.full((1, 1), acc_ref[0, 0] / count, jnp.float32)


def _sc_gather2(data_f, idx_f, data_b, idx_b):
    num_indices = idx_f.shape[1]
    width = data_f.shape[1]
    window = 128
    mesh = plsc.VectorSubcoreMesh(core_axis_name="core",
                                  subcore_axis_name="subcore")
    otype = jax.ShapeDtypeStruct((num_indices, width), data_f.dtype)

    @pl.kernel(out_type=(otype, otype), mesh=mesh)
    def gather_kernel(xf_hbm, if_hbm, xb_hbm, ib_hbm, of_hbm, ob_hbm):
        def body(if_vmem, ib_vmem, of_vmem, ob_vmem):
            pltpu.sync_copy(xf_hbm.at[if_vmem.at[0]], of_vmem)
            pltpu.sync_copy(xb_hbm.at[ib_vmem.at[0]], ob_vmem)

        idx_spec = pl.BlockSpec((1, window), index_map=lambda i: (0, i))
        out_spec = pl.BlockSpec((window, width), index_map=lambda i: (i, 0))
        pltpu.emit_pipeline(
            body,
            grid=(num_indices // window,),
            in_specs=[idx_spec, idx_spec],
            out_specs=[out_spec, out_spec],
            core_axis_name=("core", "subcore"),
            dimension_semantics=(pltpu.PARALLEL,),
        )(if_hbm, ib_hbm, of_hbm, ob_hbm)

    return gather_kernel(data_f, idx_f, data_b, idx_b)


def kernel(predict_pc, gt_pc):
    B, C, M = predict_pc.shape
    N = gt_pc.shape[2]
    TM = 1024
    n_row_tiles = M // TM

    pt = jnp.transpose(predict_pc, (0, 2, 1))
    gt = jnp.transpose(gt_pc, (0, 2, 1))
    pth = pt.astype(jnp.bfloat16)
    gh2x = (gt_pc * 2.0).astype(jnp.bfloat16)

    idx_pg, idx_gp = pl.pallas_call(
        functools.partial(_search_kernel, n_row_tiles=n_row_tiles,
                          tm=TM, n=N, m=M),
        grid=(B, n_row_tiles),
        in_specs=[
            pl.BlockSpec((1, TM, C), lambda b, i: (b, i, 0)),
            pl.BlockSpec((1, C, N), lambda b, i: (b, 0, 0)),
            pl.BlockSpec((1, TM, C), lambda b, i: (b, i, 0)),
            pl.BlockSpec((1, C, N), lambda b, i: (b, 0, 0)),
        ],
        out_specs=[
            pl.BlockSpec((1, TM, 1), lambda b, i: (b, i, 0)),
            pl.BlockSpec((1, 1, N), lambda b, i: (b, 0, 0)),
        ],
        out_shape=[
            jax.ShapeDtypeStruct((B, M, 1), jnp.int32),
            jax.ShapeDtypeStruct((B, 1, N), jnp.int32),
        ],
        scratch_shapes=[
            pltpu.VMEM((1, N), jnp.float32),
            pltpu.VMEM((1, N), jnp.int32),
        ],
        compiler_params=pltpu.CompilerParams(
            dimension_semantics=("parallel", "arbitrary")),
    )(pt, gt_pc, pth, gh2x)

    pad = ((0, 0), (0, 125))
    pt_flat = pt.reshape(B * M, C)
    gt_flat = gt.reshape(B * N, C)
    gt_rows = jnp.pad(gt_flat, pad)
    pt_rows = jnp.pad(pt_flat, pad)

    sel_f, sel_b = _sc_gather2(
        gt_rows, idx_pg.reshape(1, B * M),
        pt_rows, idx_gp.reshape(1, B * N))

    nsteps = 8
    TR = (B * M) // nsteps
    out = pl.pallas_call(
        functools.partial(_reduce_kernel, count=float(B * M), nsteps=nsteps),
        grid=(nsteps,),
        in_specs=[
            pl.BlockSpec((TR, 128), lambda k: (k, 0)),
            pl.BlockSpec((TR, C), lambda k: (k, 0)),
            pl.BlockSpec((TR, 128), lambda k: (k, 0)),
            pl.BlockSpec((TR, C), lambda k: (k, 0)),
        ],
        out_specs=pl.BlockSpec((1, 1), lambda k: (0, 0)),
        out_shape=jax.ShapeDtypeStruct((1, 1), jnp.float32),
        scratch_shapes=[pltpu.SMEM((1, 1), jnp.float32)],
    )(sel_f, pt_flat, sel_b, gt_flat)
    return out[0, 0]


# speedup vs baseline: 1.0118x; 1.0118x over previous
import functools

import jax
import jax.numpy as jnp
from jax.experimental import pallas as pl
from jax.experimental.pallas import tpu as pltpu
from jax.experimental.pallas import tpu_sc as plsc

def _search_kernel(pt_ref, g_ref, pth_ref, gh2x_ref,
                   idx_pg_ref, idx_gp_ref, colmin_ref, colidx_ref,
                   *, n_row_tiles, tm, n, m):
    b = pl.program_id(0)
    i = pl.program_id(1)

    p = pt_ref[0]
    g = g_ref[0]
    px = p[:, 0:1]
    py = p[:, 1:2]
    pz = p[:, 2:3]
    gx = g[0:1, :]
    gy = g[1:2, :]
    gz = g[2:3, :]

    p2 = px * px + py * py + pz * pz
    g2 = gx * gx + gy * gy + gz * gz
    s = p2 + g2

    cross2_sel = jnp.dot(pth_ref[0], gh2x_ref[0],
                         preferred_element_type=jnp.float32)
    d_sel = s - cross2_sel

    rowidx = jnp.argmin(d_sel, axis=1).astype(jnp.int32)[:, None]
    idx_pg_ref[0] = rowidx + b * n

    tile_colmin = jnp.min(d_sel, axis=0, keepdims=True)
    tile_colidx = (jnp.argmin(d_sel, axis=0).astype(jnp.int32)[None, :]
                   + i * tm)

    @pl.when(i == 0)
    def _init_col():
        colmin_ref[...] = tile_colmin
        colidx_ref[...] = tile_colidx

    @pl.when(i > 0)
    def _update_col():
        better = tile_colmin < colmin_ref[...]
        colmin_ref[...] = jnp.where(better, tile_colmin, colmin_ref[...])
        colidx_ref[...] = jnp.where(better, tile_colidx, colidx_ref[...])

    @pl.when(i == n_row_tiles - 1)
    def _finish_batch():
        idx_gp_ref[0] = colidx_ref[...] + b * m


def _reduce_kernel(self_f_ref, qf_ref, sel_b_ref, qb_ref, out_ref, acc_ref,
                   *, count, nsteps):
    k = pl.program_id(0)
    diff_f = self_f_ref[...][:, 0:3] - qf_ref[...]
    diff_b = sel_b_ref[...][:, 0:3] - qb_ref[...]
    rs_f = jnp.sum(diff_f * diff_f, axis=1, keepdims=True)
    rs_b = jnp.sum(diff_b * diff_b, axis=1, keepdims=True)
    part = (jnp.sum(jnp.sqrt(rs_f + 1e-8)) + jnp.sum(jnp.sqrt(rs_b + 1e-8)))

    @pl.when(k == 0)
    def _init():
        acc_ref[0, 0] = 0.0

    acc_ref[0, 0] += part

    @pl.when(k == nsteps - 1)
    def _finish():
        out_ref[...] = jnp.full((1, 1), acc_ref[0, 0] / count, jnp.float32)


def _sc_gather2(data_f, idx_f, data_b, idx_b):
    num_indices = idx_f.shape[1]
    width = data_f.shape[1]
    window = 128
    mesh = plsc.VectorSubcoreMesh(core_axis_name="core",
                                  subcore_axis_name="subcore")
    otype = jax.ShapeDtypeStruct((num_indices, width), data_f.dtype)

    @pl.kernel(out_type=(otype, otype), mesh=mesh)
    def gather_kernel(xf_hbm, if_hbm, xb_hbm, ib_hbm, of_hbm, ob_hbm):
        def body(if_vmem, ib_vmem, of_vmem, ob_vmem):
            pltpu.sync_copy(xf_hbm.at[if_vmem.at[0]], of_vmem)
            pltpu.sync_copy(xb_hbm.at[ib_vmem.at[0]], ob_vmem)

        idx_spec = pl.BlockSpec((1, window), index_map=lambda i: (0, i))
        out_spec = pl.BlockSpec((window, width), index_map=lambda i: (i, 0))
        pltpu.emit_pipeline(
            body,
            grid=(num_indices // window,),
            in_specs=[idx_spec, idx_spec],
            out_specs=[out_spec, out_spec],
            core_axis_name=("core", "subcore"),
            dimension_semantics=(pltpu.PARALLEL,),
        )(if_hbm, ib_hbm, of_hbm, ob_hbm)

    return gather_kernel(data_f, idx_f, data_b, idx_b)


def kernel(predict_pc, gt_pc):
    B, C, M = predict_pc.shape
    N = gt_pc.shape[2]
    TM = 2048
    n_row_tiles = M // TM

    pt = jnp.transpose(predict_pc, (0, 2, 1))
    gt = jnp.transpose(gt_pc, (0, 2, 1))
    pth = pt.astype(jnp.bfloat16)
    gh2x = (gt_pc * 2.0).astype(jnp.bfloat16)

    idx_pg, idx_gp = pl.pallas_call(
        functools.partial(_search_kernel, n_row_tiles=n_row_tiles,
                          tm=TM, n=N, m=M),
        grid=(B, n_row_tiles),
        in_specs=[
            pl.BlockSpec((1, TM, C), lambda b, i: (b, i, 0)),
            pl.BlockSpec((1, C, N), lambda b, i: (b, 0, 0)),
            pl.BlockSpec((1, TM, C), lambda b, i: (b, i, 0)),
            pl.BlockSpec((1, C, N), lambda b, i: (b, 0, 0)),
        ],
        out_specs=[
            pl.BlockSpec((1, TM, 1), lambda b, i: (b, i, 0)),
            pl.BlockSpec((1, 1, N), lambda b, i: (b, 0, 0)),
        ],
        out_shape=[
            jax.ShapeDtypeStruct((B, M, 1), jnp.int32),
            jax.ShapeDtypeStruct((B, 1, N), jnp.int32),
        ],
        scratch_shapes=[
            pltpu.VMEM((1, N), jnp.float32),
            pltpu.VMEM((1, N), jnp.int32),
        ],
        compiler_params=pltpu.CompilerParams(
            dimension_semantics=("parallel", "arbitrary")),
    )(pt, gt_pc, pth, gh2x)

    pad = ((0, 0), (0, 125))
    pt_flat = pt.reshape(B * M, C)
    gt_flat = gt.reshape(B * N, C)
    gt_rows = jnp.pad(gt_flat, pad)
    pt_rows = jnp.pad(pt_flat, pad)

    sel_f, sel_b = _sc_gather2(
        gt_rows, idx_pg.reshape(1, B * M),
        pt_rows, idx_gp.reshape(1, B * N))

    nsteps = 8
    TR = (B * M) // nsteps
    out = pl.pallas_call(
        functools.partial(_reduce_kernel, count=float(B * M), nsteps=nsteps),
        grid=(nsteps,),
        in_specs=[
            pl.BlockSpec((TR, 128), lambda k: (k, 0)),
            pl.BlockSpec((TR, C), lambda k: (k, 0)),
            pl.BlockSpec((TR, 128), lambda k: (k, 0)),
            pl.BlockSpec((TR, C), lambda k: (k, 0)),
        ],
        out_specs=pl.BlockSpec((1, 1), lambda k: (0, 0)),
        out_shape=jax.ShapeDtypeStruct((1, 1), jnp.float32),
        scratch_shapes=[pltpu.SMEM((1, 1), jnp.float32)],
    )(sel_f, pt_flat, sel_b, gt_flat)
    return out[0, 0]
